# baseline (speedup 1.0000x reference)
import functools

import jax
import jax.numpy as jnp
from jax import lax
from jax.experimental import pallas as pl
from jax.experimental.pallas import tpu as pltpu
from jax.experimental.pallas import tpu_sc as plsc

NC = 2
NS = 16
NW = NC * NS
L = 16


def _make_sc_kernel(B, F, V):
    rows_pt = B // NW
    assert rows_pt % 128 == 0
    cpf = rows_pt // 128
    nchunk = F * cpf
    groups = rows_pt // L

    mesh = plsc.VectorSubcoreMesh(core_axis_name="c", subcore_axis_name="s")

    slab = (V // NS) & ~63
    tail0 = V - NS * slab

    @functools.partial(
        pl.kernel,
        out_type=jax.ShapeDtypeStruct((B,), jnp.float32),
        mesh=mesh,
        scratch_types=[
            pltpu.VMEM_SHARED((V,), jnp.float32),
            pltpu.VMEM((slab // 8,), jnp.float32),
            pltpu.VMEM((slab // 8,), jnp.float32),
            pltpu.VMEM((F, rows_pt), jnp.int32),
            pltpu.VMEM((F, rows_pt), jnp.float32),
            pltpu.VMEM((F, rows_pt), jnp.float32),
            pltpu.VMEM((L,), jnp.float32),
            pltpu.VMEM((rows_pt,), jnp.float32),
            pltpu.SemaphoreType.DMA,
        ],
    )
    def sc_kernel(w_hbm, ids_hbm, vals_hbm, bias_hbm, out_hbm,
                  spw, bounce, bounce2, idx_v, vals_v, wg_v, bias_v, out_v, sem):
        cid = lax.axis_index("c")
        sid = lax.axis_index("s")
        wid = sid * NC + cid
        b0 = wid * rows_pt

        s0 = sid * slab
        q = slab // 8
        bufs = (bounce, bounce2)
        sem2 = sem
        pltpu.async_copy(w_hbm.at[pl.ds(s0, q)], bounce, sem)
        pltpu.async_copy(w_hbm.at[pl.ds(s0 + q, q)], bounce2, sem2)
        pltpu.sync_copy(ids_hbm.at[:, pl.ds(b0, rows_pt)], idx_v)
        pltpu.sync_copy(vals_hbm.at[:, pl.ds(b0, rows_pt)], vals_v)
        pltpu.sync_copy(bias_hbm, bias_v)
        for r in range(8):
            buf = bufs[r % 2]
            pltpu.make_async_copy(w_hbm.at[pl.ds(s0 + r * q, q)], buf, sem).wait()
            if r + 2 < 8:
                pltpu.async_copy(w_hbm.at[pl.ds(s0 + (r + 2) * q, q)], buf, sem)
            pltpu.sync_copy(buf, spw.at[pl.ds(s0 + r * q, q)])

        if tail0:
            @pl.when(sid == 0)
            def _():
                pltpu.async_copy(w_hbm.at[pl.ds(NS * slab, tail0)],
                                 bounce.at[pl.ds(0, tail0)], sem).wait()
                pltpu.sync_copy(bounce.at[pl.ds(0, tail0)],
                                spw.at[pl.ds(NS * slab, tail0)])

        plsc.subcore_barrier()

        def fire(j, carry):
            f = j // cpf
            c = (j % cpf) * 128
            pltpu.async_copy(
                spw.at[idx_v.at[f, pl.ds(c, 128)]],
                wg_v.at[f, pl.ds(c, 128)], sem)
            return carry

        lax.fori_loop(0, nchunk, fire, 0)

        bias_vec = bias_v[...]

        def init(g, carry):
            out_v[pl.ds(g * L, L)] = bias_vec
            return carry

        lax.fori_loop(0, groups, init, 0)

        def drain_compute(j, carry):
            f = j // cpf
            c = (j % cpf) * 128
            pltpu.make_async_copy(
                spw.at[idx_v.at[f, pl.ds(c, 128)]],
                wg_v.at[f, pl.ds(c, 128)], sem).wait()
            for g in range(128 // L):
                o = c + g * L
                out_v[pl.ds(o, L)] = (out_v[pl.ds(o, L)]
                                      + wg_v[f, pl.ds(o, L)] * vals_v[f, pl.ds(o, L)])
            return carry

        lax.fori_loop(0, nchunk, drain_compute, 0)

        pltpu.sync_copy(out_v, out_hbm.at[pl.ds(b0, rows_pt)])

    return sc_kernel


def kernel(ids, vals, weight, bias):
    B, F = ids.shape
    V = weight.shape[0]
    pad = (-V) % 1024
    w_flat = jnp.pad(weight, ((0, pad), (0, 0))).reshape(-1)
    ids_t = ids.T
    vals_t = vals.T
    bias16 = jnp.broadcast_to(bias.astype(jnp.float32), (L,))
    sc = _make_sc_kernel(B, F, V + pad)
    return sc(w_flat, ids_t, vals_t, bias16)


# speedup vs baseline: 1.0701x; 1.0701x over previous
import functools

import jax
import jax.numpy as jnp
from jax import lax
from jax.experimental import pallas as pl
from jax.experimental.pallas import tpu as pltpu
from jax.experimental.pallas import tpu_sc as plsc

NC = 2
NS = 16
NW = NC * NS
L = 16


def _make_sc_kernel(B, F, V):
    rows_pt = B // NW
    assert rows_pt % 128 == 0
    cpf = rows_pt // 128
    nchunk = F * cpf
    groups = rows_pt // L

    mesh = plsc.VectorSubcoreMesh(core_axis_name="c", subcore_axis_name="s")

    slab = (V // NS) & ~63
    tail0 = V - NS * slab

    @functools.partial(
        pl.kernel,
        out_type=jax.ShapeDtypeStruct((B,), jnp.float32),
        mesh=mesh,
        scratch_types=[
            pltpu.VMEM_SHARED((V,), jnp.float32),
            pltpu.VMEM((slab // 8,), jnp.float32),
            pltpu.VMEM((slab // 8,), jnp.float32),
            pltpu.VMEM((F, rows_pt), jnp.int32),
            pltpu.VMEM((F, rows_pt), jnp.float32),
            pltpu.VMEM((F, rows_pt), jnp.float32),
            pltpu.VMEM((L,), jnp.float32),
            pltpu.VMEM((rows_pt,), jnp.float32),
            pltpu.SemaphoreType.DMA,
        ],
    )
    def sc_kernel(w_hbm, ids_hbm, vals_hbm, bias_hbm, out_hbm,
                  spw, bounce, bounce2, idx_v, vals_v, wg_v, bias_v, out_v, sem):
        cid = lax.axis_index("c")
        sid = lax.axis_index("s")
        wid = sid * NC + cid
        b0 = wid * rows_pt

        s0 = sid * slab
        q = slab // 8
        bufs = (bounce, bounce2)
        sem2 = sem
        pltpu.async_copy(w_hbm.at[pl.ds(s0, q)], bounce, sem)
        pltpu.async_copy(w_hbm.at[pl.ds(s0 + q, q)], bounce2, sem2)
        pltpu.sync_copy(ids_hbm.at[:, pl.ds(b0, rows_pt)], idx_v)
        pltpu.sync_copy(vals_hbm.at[:, pl.ds(b0, rows_pt)], vals_v)
        pltpu.sync_copy(bias_hbm, bias_v)
        for r in range(8):
            buf = bufs[r % 2]
            pltpu.make_async_copy(w_hbm.at[pl.ds(s0 + r * q, q)], buf, sem).wait()
            if r + 2 < 8:
                pltpu.async_copy(w_hbm.at[pl.ds(s0 + (r + 2) * q, q)], buf, sem)
            pltpu.sync_copy(buf, spw.at[pl.ds(s0 + r * q, q)])

        if tail0:
            @pl.when(sid == 0)
            def _():
                pltpu.async_copy(w_hbm.at[pl.ds(NS * slab, tail0)],
                                 bounce.at[pl.ds(0, tail0)], sem).wait()
                pltpu.sync_copy(bounce.at[pl.ds(0, tail0)],
                                spw.at[pl.ds(NS * slab, tail0)])

        plsc.subcore_barrier()

        def fire(f, carry):
            for k in range(cpf):
                c = k * 128
                pltpu.async_copy(
                    spw.at[idx_v.at[f, pl.ds(c, 128)]],
                    wg_v.at[f, pl.ds(c, 128)], sem)
            return carry

        lax.fori_loop(0, F, fire, 0)

        def drain(f, carry):
            for k in range(cpf):
                c = k * 128
                pltpu.make_async_copy(
                    spw.at[idx_v.at[f, pl.ds(c, 128)]],
                    wg_v.at[f, pl.ds(c, 128)], sem).wait()
            return carry

        lax.fori_loop(0, F, drain, 0)

        bias_vec = bias_v[...]

        def group(g, carry):
            o = g * L
            acc = bias_vec
            for f in range(F):
                acc = acc + wg_v[f, pl.ds(o, L)] * vals_v[f, pl.ds(o, L)]
            out_v[pl.ds(o, L)] = acc
            return carry

        lax.fori_loop(0, groups, group, 0)

        pltpu.sync_copy(out_v, out_hbm.at[pl.ds(b0, rows_pt)])

    return sc_kernel


def kernel(ids, vals, weight, bias):
    B, F = ids.shape
    V = weight.shape[0]
    pad = (-V) % 1024
    w_flat = jnp.pad(weight, ((0, pad), (0, 0))).reshape(-1)
    ids_t = ids.T
    vals_t = vals.T
    bias16 = jnp.broadcast_to(bias.astype(jnp.float32), (L,))
    sc = _make_sc_kernel(B, F, V + pad)
    return sc(w_flat, ids_t, vals_t, bias16)
